# Initial kernel scaffold; baseline (speedup 1.0000x reference)
import jax
import jax.numpy as jnp
from jax.experimental import pallas as pl


def kernel(x, h0, c0, emb, W_ih0, W_hh0, b_ih0, b_hh0, W_ih1, W_hh1, b_ih1, b_hh1, W_ih2, W_hh2, b_ih2, b_hh2, fc_W, fc_b):
    raise NotImplementedError("write your pallas kernel here")


# speedup vs baseline: 1.1389x; 1.1389x over previous
import functools

import jax
import jax.numpy as jnp
from jax import lax
from jax.experimental import pallas as pl
from jax.experimental.pallas import tpu as pltpu
from jax.experimental.pallas import tpu_sc as plsc

_T, _B, _D, _L = 20, 32, 128, 3
_TB = _T * _B
_G4 = 4 * _D


@functools.lru_cache(maxsize=None)
def _make_sc_gather(vocab, d, n_idx):
    info = plsc.get_sparse_core_info()
    nc, ns = info.num_cores, info.num_subcores
    nw = nc * ns
    b_per_w = 32
    n_active = n_idx // b_per_w
    assert n_idx % b_per_w == 0 and n_active <= nw
    mesh = plsc.VectorSubcoreMesh(core_axis_name="c", subcore_axis_name="s")

    @functools.partial(
        pl.kernel,
        mesh=mesh,
        out_type=jax.ShapeDtypeStruct((n_idx, d), jnp.float32),
        scratch_types=[
            pltpu.VMEM((b_per_w,), jnp.int32),
            pltpu.VMEM((b_per_w, d), jnp.float32),
            pltpu.SemaphoreType.DMA,
        ],
    )
    def gather_k(table_hbm, idx_hbm, out_hbm, idx_v, rows_v, sem):
        wid = lax.axis_index("s") * nc + lax.axis_index("c")

        @pl.when(wid < n_active)
        def _():
            base = wid * b_per_w
            pltpu.sync_copy(idx_hbm.at[pl.ds(base, b_per_w)], idx_v)
            pltpu.async_copy(table_hbm.at[idx_v], rows_v, sem).wait()
            pltpu.sync_copy(rows_v, out_hbm.at[pl.ds(base, b_per_w)])

    return gather_k


def _lstm_body(x_ref, h0_ref, c0_ref,
               wih0, whh0, bih0, bhh0,
               wih1, whh1, bih1, bhh1,
               wih2, whh2, bih2, bhh2,
               ys_ref, ht_ref, ct_ref, gx_ref):
    params = ((wih0, whh0, bih0, bhh0),
              (wih1, whh1, bih1, bhh1),
              (wih2, whh2, bih2, bhh2))
    dn = (((1,), (1,)), ((), ()))
    for l in range(_L):
        wih, whh, bih, bhh = params[l]
        src = x_ref if l == 0 else ys_ref
        gx_ref[:] = (
            lax.dot_general(src[:], wih[:], dn, preferred_element_type=jnp.float32)
            + bih[:] + bhh[:]
        )
        h = h0_ref[l]
        c = c0_ref[l]
        whh_v = whh[:]
        for t in range(_T):
            gates = gx_ref[t * _B:(t + 1) * _B, :] + lax.dot_general(
                h, whh_v, dn, preferred_element_type=jnp.float32)
            i = jax.nn.sigmoid(gates[:, :_D])
            f = jax.nn.sigmoid(gates[:, _D:2 * _D])
            g = jnp.tanh(gates[:, 2 * _D:3 * _D])
            o = jax.nn.sigmoid(gates[:, 3 * _D:])
            c = f * c + i * g
            h = o * jnp.tanh(c)
            ys_ref[t * _B:(t + 1) * _B, :] = h
        ht_ref[l] = h
        ct_ref[l] = c


def _run_lstm(xs, h0, c0, ws):
    out_shapes = (
        jax.ShapeDtypeStruct((_TB, _D), jnp.float32),
        jax.ShapeDtypeStruct((_L, _B, _D), jnp.float32),
        jax.ShapeDtypeStruct((_L, _B, _D), jnp.float32),
    )
    return pl.pallas_call(
        _lstm_body,
        out_shape=out_shapes,
        scratch_shapes=[pltpu.VMEM((_TB, _G4), jnp.float32)],
    )(xs, h0, c0, *ws)


_V_TILE = 1024


def _proj_body(a_ref, w_ref, b_ref, o_ref):
    o_ref[:] = lax.dot_general(
        a_ref[:], w_ref[:], (((1,), (1,)), ((), ())),
        preferred_element_type=jnp.float32) + b_ref[:]


def _run_proj(ys, fc_w, fc_b2d, vocab):
    grid = (pl.cdiv(vocab, _V_TILE),)
    return pl.pallas_call(
        _proj_body,
        grid=grid,
        in_specs=[
            pl.BlockSpec((_TB, _D), lambda i: (0, 0)),
            pl.BlockSpec((_V_TILE, _D), lambda i: (i, 0)),
            pl.BlockSpec((1, _V_TILE), lambda i: (0, i)),
        ],
        out_specs=pl.BlockSpec((_TB, _V_TILE), lambda i: (0, i)),
        out_shape=jax.ShapeDtypeStruct((_TB, vocab), jnp.float32),
        compiler_params=pltpu.CompilerParams(
            dimension_semantics=("arbitrary",)),
    )(ys, fc_w, fc_b2d)


def kernel(x, h0, c0, emb,
           W_ih0, W_hh0, b_ih0, b_hh0,
           W_ih1, W_hh1, b_ih1, b_hh1,
           W_ih2, W_hh2, b_ih2, b_hh2,
           fc_W, fc_b):
    vocab = emb.shape[0]
    idx = x.reshape(_TB)
    gathered = _make_sc_gather(vocab, _D, _TB)(emb, idx)

    ws = (W_ih0, W_hh0, b_ih0.reshape(1, _G4), b_hh0.reshape(1, _G4),
          W_ih1, W_hh1, b_ih1.reshape(1, _G4), b_hh1.reshape(1, _G4),
          W_ih2, W_hh2, b_ih2.reshape(1, _G4), b_hh2.reshape(1, _G4))
    ys, ht, ct = _run_lstm(gathered, h0, c0, ws)

    logits = _run_proj(ys, fc_W, fc_b.reshape(1, vocab), vocab)
    return logits.reshape(_T, _B, vocab), (ht, ct)


# speedup vs baseline: 1.3827x; 1.2140x over previous
import functools

import jax
import jax.numpy as jnp
from jax import lax
from jax.experimental import pallas as pl
from jax.experimental.pallas import tpu as pltpu
from jax.experimental.pallas import tpu_sc as plsc

_T, _B, _D, _L = 20, 32, 128, 3
_TB = _T * _B
_G4 = 4 * _D


@functools.lru_cache(maxsize=None)
def _make_sc_gather(vocab, d, n_idx):
    info = plsc.get_sparse_core_info()
    nc, ns = info.num_cores, info.num_subcores
    nw = nc * ns
    b_per_w = 32
    n_active = n_idx // b_per_w
    assert n_idx % b_per_w == 0 and n_active <= nw
    mesh = plsc.VectorSubcoreMesh(core_axis_name="c", subcore_axis_name="s")

    @functools.partial(
        pl.kernel,
        mesh=mesh,
        out_type=jax.ShapeDtypeStruct((n_idx, d), jnp.float32),
        scratch_types=[
            pltpu.VMEM((b_per_w,), jnp.int32),
            pltpu.VMEM((b_per_w, d), jnp.float32),
            pltpu.SemaphoreType.DMA,
        ],
    )
    def gather_k(table_hbm, idx_hbm, out_hbm, idx_v, rows_v, sem):
        wid = lax.axis_index("s") * nc + lax.axis_index("c")

        @pl.when(wid < n_active)
        def _():
            base = wid * b_per_w
            pltpu.sync_copy(idx_hbm.at[pl.ds(base, b_per_w)], idx_v)
            pltpu.async_copy(table_hbm.at[idx_v], rows_v, sem).wait()
            pltpu.sync_copy(rows_v, out_hbm.at[pl.ds(base, b_per_w)])

    return gather_k


def _lstm_body(x_ref, h0_ref, c0_ref,
               wih0, whh0, bih0, bhh0,
               wih1, whh1, bih1, bhh1,
               wih2, whh2, bih2, bhh2,
               ys_ref, ht_ref, ct_ref, gx_ref):
    params = ((wih0, whh0, bih0, bhh0),
              (wih1, whh1, bih1, bhh1),
              (wih2, whh2, bih2, bhh2))
    dn = (((1,), (1,)), ((), ()))
    for l in range(_L):
        wih, whh, bih, bhh = params[l]
        src = x_ref if l == 0 else ys_ref
        gx_ref[:] = (
            lax.dot_general(src[:], wih[:], dn, preferred_element_type=jnp.float32)
            + bih[:] + bhh[:]
        )
        h = h0_ref[l]
        c = c0_ref[l]
        whh_v = whh[:]
        for t in range(_T):
            gates = gx_ref[t * _B:(t + 1) * _B, :] + lax.dot_general(
                h, whh_v, dn, preferred_element_type=jnp.float32)
            i = jax.nn.sigmoid(gates[:, :_D])
            f = jax.nn.sigmoid(gates[:, _D:2 * _D])
            g = jnp.tanh(gates[:, 2 * _D:3 * _D])
            o = jax.nn.sigmoid(gates[:, 3 * _D:])
            c = f * c + i * g
            h = o * jnp.tanh(c)
            ys_ref[t * _B:(t + 1) * _B, :] = h
        ht_ref[l] = h
        ct_ref[l] = c


def _run_lstm(xs, h0, c0, ws):
    out_shapes = (
        jax.ShapeDtypeStruct((_TB, _D), jnp.float32),
        jax.ShapeDtypeStruct((_L, _B, _D), jnp.float32),
        jax.ShapeDtypeStruct((_L, _B, _D), jnp.float32),
    )
    return pl.pallas_call(
        _lstm_body,
        out_shape=out_shapes,
        scratch_shapes=[pltpu.VMEM((_TB, _G4), jnp.float32)],
    )(xs, h0, c0, *ws)


_V_TILE = 2048


def _proj_body(a_ref, w_ref, b_ref, o_ref):
    o_ref[:] = lax.dot_general(
        a_ref[:], w_ref[:], (((1,), (1,)), ((), ())),
        preferred_element_type=jnp.float32) + b_ref[:]


def _run_proj(ys, fc_w, fc_b2d, vocab):
    grid = (pl.cdiv(vocab, _V_TILE),)
    return pl.pallas_call(
        _proj_body,
        grid=grid,
        in_specs=[
            pl.BlockSpec((_TB, _D), lambda i: (0, 0)),
            pl.BlockSpec((_V_TILE, _D), lambda i: (i, 0)),
            pl.BlockSpec((1, _V_TILE), lambda i: (0, i)),
        ],
        out_specs=pl.BlockSpec((_TB, _V_TILE), lambda i: (0, i)),
        out_shape=jax.ShapeDtypeStruct((_TB, vocab), jnp.float32),
        compiler_params=pltpu.CompilerParams(
            dimension_semantics=("arbitrary",)),
    )(ys, fc_w, fc_b2d)


def kernel(x, h0, c0, emb,
           W_ih0, W_hh0, b_ih0, b_hh0,
           W_ih1, W_hh1, b_ih1, b_hh1,
           W_ih2, W_hh2, b_ih2, b_hh2,
           fc_W, fc_b):
    vocab = emb.shape[0]
    idx = x.reshape(_TB)
    gathered = _make_sc_gather(vocab, _D, _TB)(emb, idx)

    ws = (W_ih0, W_hh0, b_ih0.reshape(1, _G4), b_hh0.reshape(1, _G4),
          W_ih1, W_hh1, b_ih1.reshape(1, _G4), b_hh1.reshape(1, _G4),
          W_ih2, W_hh2, b_ih2.reshape(1, _G4), b_hh2.reshape(1, _G4))
    ys, ht, ct = _run_lstm(gathered, h0, c0, ws)

    logits = _run_proj(ys, fc_W, fc_b.reshape(1, vocab), vocab)
    return logits.reshape(_T, _B, vocab), (ht, ct)


# speedup vs baseline: 1.4548x; 1.0522x over previous
import functools

import jax
import jax.numpy as jnp
from jax import lax
from jax.experimental import pallas as pl
from jax.experimental.pallas import tpu as pltpu
from jax.experimental.pallas import tpu_sc as plsc

_T, _B, _D, _L = 20, 32, 128, 3
_TB = _T * _B
_G4 = 4 * _D


@functools.lru_cache(maxsize=None)
def _make_sc_gather(vocab, d, n_idx):
    info = plsc.get_sparse_core_info()
    nc, ns = info.num_cores, info.num_subcores
    nw = nc * ns
    b_per_w = 32
    n_active = n_idx // b_per_w
    assert n_idx % b_per_w == 0 and n_active <= nw
    mesh = plsc.VectorSubcoreMesh(core_axis_name="c", subcore_axis_name="s")

    @functools.partial(
        pl.kernel,
        mesh=mesh,
        out_type=jax.ShapeDtypeStruct((n_idx, d), jnp.float32),
        scratch_types=[
            pltpu.VMEM((b_per_w,), jnp.int32),
            pltpu.VMEM((b_per_w, d), jnp.float32),
            pltpu.SemaphoreType.DMA,
        ],
    )
    def gather_k(table_hbm, idx_hbm, out_hbm, idx_v, rows_v, sem):
        wid = lax.axis_index("s") * nc + lax.axis_index("c")

        @pl.when(wid < n_active)
        def _():
            base = wid * b_per_w
            pltpu.sync_copy(idx_hbm.at[pl.ds(base, b_per_w)], idx_v)
            pltpu.async_copy(table_hbm.at[idx_v], rows_v, sem).wait()
            pltpu.sync_copy(rows_v, out_hbm.at[pl.ds(base, b_per_w)])

    return gather_k


def _lstm_body(x_ref, h0_ref, c0_ref,
               wih0, whh0, bih0, bhh0,
               wih1, whh1, bih1, bhh1,
               wih2, whh2, bih2, bhh2,
               ys_ref, ht_ref, ct_ref, gx_ref):
    params = ((wih0, whh0, bih0, bhh0),
              (wih1, whh1, bih1, bhh1),
              (wih2, whh2, bih2, bhh2))
    dn = (((1,), (1,)), ((), ()))
    for l in range(_L):
        wih, whh, bih, bhh = params[l]
        src = x_ref if l == 0 else ys_ref
        gx_ref[:] = (
            lax.dot_general(src[:], wih[:], dn, preferred_element_type=jnp.float32)
            + bih[:] + bhh[:]
        )
        h = h0_ref[l]
        c = c0_ref[l]
        whh_v = whh[:]
        for t in range(_T):
            gates = gx_ref[t * _B:(t + 1) * _B, :] + lax.dot_general(
                h, whh_v, dn, preferred_element_type=jnp.float32)
            i = jax.nn.sigmoid(gates[:, :_D])
            f = jax.nn.sigmoid(gates[:, _D:2 * _D])
            g = jnp.tanh(gates[:, 2 * _D:3 * _D])
            o = jax.nn.sigmoid(gates[:, 3 * _D:])
            c = f * c + i * g
            h = o * jnp.tanh(c)
            ys_ref[t * _B:(t + 1) * _B, :] = h
        ht_ref[l] = h
        ct_ref[l] = c


def _run_lstm(xs, h0, c0, ws):
    out_shapes = (
        jax.ShapeDtypeStruct((_TB, _D), jnp.float32),
        jax.ShapeDtypeStruct((_L, _B, _D), jnp.float32),
        jax.ShapeDtypeStruct((_L, _B, _D), jnp.float32),
    )
    return pl.pallas_call(
        _lstm_body,
        out_shape=out_shapes,
        scratch_shapes=[pltpu.VMEM((_TB, _G4), jnp.float32)],
    )(xs, h0, c0, *ws)


_V_TILE = 4096


def _proj_body(a_ref, w_ref, b_ref, o_ref):
    o_ref[:] = lax.dot_general(
        a_ref[:], w_ref[:], (((1,), (1,)), ((), ())),
        preferred_element_type=jnp.float32) + b_ref[:]


def _run_proj(ys, fc_w, fc_b2d, vocab):
    grid = (pl.cdiv(vocab, _V_TILE),)
    return pl.pallas_call(
        _proj_body,
        grid=grid,
        in_specs=[
            pl.BlockSpec((_TB, _D), lambda i: (0, 0)),
            pl.BlockSpec((_V_TILE, _D), lambda i: (i, 0)),
            pl.BlockSpec((1, _V_TILE), lambda i: (0, i)),
        ],
        out_specs=pl.BlockSpec((_TB, _V_TILE), lambda i: (0, i)),
        out_shape=jax.ShapeDtypeStruct((_TB, vocab), jnp.float32),
        compiler_params=pltpu.CompilerParams(
            dimension_semantics=("arbitrary",)),
    )(ys, fc_w, fc_b2d)


def kernel(x, h0, c0, emb,
           W_ih0, W_hh0, b_ih0, b_hh0,
           W_ih1, W_hh1, b_ih1, b_hh1,
           W_ih2, W_hh2, b_ih2, b_hh2,
           fc_W, fc_b):
    vocab = emb.shape[0]
    idx = x.reshape(_TB)
    gathered = _make_sc_gather(vocab, _D, _TB)(emb, idx)

    ws = (W_ih0, W_hh0, b_ih0.reshape(1, _G4), b_hh0.reshape(1, _G4),
          W_ih1, W_hh1, b_ih1.reshape(1, _G4), b_hh1.reshape(1, _G4),
          W_ih2, W_hh2, b_ih2.reshape(1, _G4), b_hh2.reshape(1, _G4))
    ys, ht, ct = _run_lstm(gathered, h0, c0, ws)

    logits = _run_proj(ys, fc_W, fc_b.reshape(1, vocab), vocab)
    return logits.reshape(_T, _B, vocab), (ht, ct)


# speedup vs baseline: 1.4772x; 1.0154x over previous
import functools

import jax
import jax.numpy as jnp
from jax import lax
from jax.experimental import pallas as pl
from jax.experimental.pallas import tpu as pltpu
from jax.experimental.pallas import tpu_sc as plsc

_T, _B, _D, _L = 20, 32, 128, 3
_TB = _T * _B
_G4 = 4 * _D


@functools.lru_cache(maxsize=None)
def _make_sc_gather(vocab, d, n_idx):
    info = plsc.get_sparse_core_info()
    nc, ns = info.num_cores, info.num_subcores
    nw = nc * ns
    b_per_w = 32
    n_active = n_idx // b_per_w
    assert n_idx % b_per_w == 0 and n_active <= nw
    mesh = plsc.VectorSubcoreMesh(core_axis_name="c", subcore_axis_name="s")

    @functools.partial(
        pl.kernel,
        mesh=mesh,
        out_type=jax.ShapeDtypeStruct((n_idx, d), jnp.float32),
        scratch_types=[
            pltpu.VMEM((b_per_w,), jnp.int32),
            pltpu.VMEM((b_per_w, d), jnp.float32),
            pltpu.SemaphoreType.DMA,
        ],
    )
    def gather_k(table_hbm, idx_hbm, out_hbm, idx_v, rows_v, sem):
        wid = lax.axis_index("s") * nc + lax.axis_index("c")

        @pl.when(wid < n_active)
        def _():
            base = wid * b_per_w
            pltpu.sync_copy(idx_hbm.at[pl.ds(base, b_per_w)], idx_v)
            pltpu.async_copy(table_hbm.at[idx_v], rows_v, sem).wait()
            pltpu.sync_copy(rows_v, out_hbm.at[pl.ds(base, b_per_w)])

    return gather_k


def _lstm_body(x_ref, h0_ref, c0_ref,
               wih0, whh0, bih0, bhh0,
               wih1, whh1, bih1, bhh1,
               wih2, whh2, bih2, bhh2,
               ys_ref, ht_ref, ct_ref, gx_ref):
    params = ((wih0, whh0, bih0, bhh0),
              (wih1, whh1, bih1, bhh1),
              (wih2, whh2, bih2, bhh2))
    dn = (((1,), (1,)), ((), ()))
    for l in range(_L):
        wih, whh, bih, bhh = params[l]
        src = x_ref if l == 0 else ys_ref
        gx_ref[:] = (
            lax.dot_general(src[:], wih[:], dn, preferred_element_type=jnp.float32)
            + bih[:] + bhh[:]
        )
        h = h0_ref[l]
        c = c0_ref[l]
        whh_v = whh[:]
        for t in range(_T):
            gates = gx_ref[t * _B:(t + 1) * _B, :] + lax.dot_general(
                h, whh_v, dn, preferred_element_type=jnp.float32)
            i = jax.nn.sigmoid(gates[:, :_D])
            f = jax.nn.sigmoid(gates[:, _D:2 * _D])
            g = jnp.tanh(gates[:, 2 * _D:3 * _D])
            o = jax.nn.sigmoid(gates[:, 3 * _D:])
            c = f * c + i * g
            h = o * jnp.tanh(c)
            ys_ref[t * _B:(t + 1) * _B, :] = h
        ht_ref[l] = h
        ct_ref[l] = c


def _run_lstm(xs, h0, c0, ws):
    out_shapes = (
        jax.ShapeDtypeStruct((_TB, _D), jnp.float32),
        jax.ShapeDtypeStruct((_L, _B, _D), jnp.float32),
        jax.ShapeDtypeStruct((_L, _B, _D), jnp.float32),
    )
    return pl.pallas_call(
        _lstm_body,
        out_shape=out_shapes,
        scratch_shapes=[pltpu.VMEM((_TB, _G4), jnp.float32)],
    )(xs, h0, c0, *ws)


_V_TILE = 8192


def _proj_body(a_ref, w_ref, b_ref, o_ref):
    o_ref[:] = lax.dot_general(
        a_ref[:], w_ref[:], (((1,), (1,)), ((), ())),
        preferred_element_type=jnp.float32) + b_ref[:]


def _run_proj(ys, fc_w, fc_b2d, vocab):
    grid = (pl.cdiv(vocab, _V_TILE),)
    return pl.pallas_call(
        _proj_body,
        grid=grid,
        in_specs=[
            pl.BlockSpec((_TB, _D), lambda i: (0, 0)),
            pl.BlockSpec((_V_TILE, _D), lambda i: (i, 0)),
            pl.BlockSpec((1, _V_TILE), lambda i: (0, i)),
        ],
        out_specs=pl.BlockSpec((_TB, _V_TILE), lambda i: (0, i)),
        out_shape=jax.ShapeDtypeStruct((_TB, vocab), jnp.float32),
        compiler_params=pltpu.CompilerParams(
            dimension_semantics=("arbitrary",)),
    )(ys, fc_w, fc_b2d)


def kernel(x, h0, c0, emb,
           W_ih0, W_hh0, b_ih0, b_hh0,
           W_ih1, W_hh1, b_ih1, b_hh1,
           W_ih2, W_hh2, b_ih2, b_hh2,
           fc_W, fc_b):
    vocab = emb.shape[0]
    idx = x.reshape(_TB)
    gathered = _make_sc_gather(vocab, _D, _TB)(emb, idx)

    ws = (W_ih0, W_hh0, b_ih0.reshape(1, _G4), b_hh0.reshape(1, _G4),
          W_ih1, W_hh1, b_ih1.reshape(1, _G4), b_hh1.reshape(1, _G4),
          W_ih2, W_hh2, b_ih2.reshape(1, _G4), b_hh2.reshape(1, _G4))
    ys, ht, ct = _run_lstm(gathered, h0, c0, ws)

    logits = _run_proj(ys, fc_W, fc_b.reshape(1, vocab), vocab)
    return logits.reshape(_T, _B, vocab), (ht, ct)
